# baseline (speedup 1.0000x reference)
import jax
import jax.numpy as jnp
from jax.experimental import pallas as pl
from jax.experimental.pallas import tpu as pltpu

ALPHA = 16.0
N = 4096
R = 256
G = N // R
CL = 1024
NC = N // CL
LANES = 128


def _fold128(x, combine):
    acc = x[:, 0:LANES]
    for k in range(1, x.shape[1] // LANES):
        acc = combine(acc, x[:, k * LANES:(k + 1) * LANES])
    return acc


def _nca_body(x_row_ref, t_row_ref, x_col_ref, t_col_ref,
              loss_ref, prec_ref, mps_ref, mns_ref, sim_scr):
    i = pl.program_id(0)

    x_row = x_row_ref[...]
    t_row = t_row_ref[...]

    thr128 = None
    for c in range(NC):
        cs = slice(c * CL, (c + 1) * CL)
        sim = jnp.abs(x_col_ref[:, cs] - x_row)
        sim_scr[:, cs] = sim
        pos = t_col_ref[:, cs] == t_row
        excl = pos & (sim >= 1.0)
        v = jnp.where(excl, -1.0, sim)
        m = _fold128(v, jnp.maximum)
        thr128 = m if thr128 is None else jnp.maximum(thr128, m)
    thr = jnp.max(thr128, axis=1, keepdims=True)
    min1thr = jnp.minimum(thr, 1.0)

    a_sb = jnp.zeros((R, LANES), jnp.float32)
    a_pn = jnp.zeros((R, LANES), jnp.float32)
    a_pv = jnp.zeros((R, LANES), jnp.float32)
    for c in range(NC):
        cs = slice(c * CL, (c + 1) * CL)
        sim = sim_scr[:, cs]
        pos = t_col_ref[:, cs] == t_row
        lt1 = sim < 1.0
        below = sim < thr
        w = jnp.exp(-ALPHA * sim)
        pv_m = pos & lt1
        pn_m = pv_m & below
        selb = below & jnp.logical_not(pos & jnp.logical_not(lt1))
        a_sb = a_sb + _fold128(jnp.where(selb, w, 0.0), jnp.add)
        a_pn = a_pn + _fold128(jnp.where(pn_m, w, 0.0), jnp.add)
        a_pv = a_pv + _fold128(jnp.where(pv_m, w, 0.0), jnp.add)

    p_neig = jnp.sum(a_pn, axis=1, keepdims=True)
    p_valid = jnp.sum(a_pv, axis=1, keepdims=True)
    s_sb = jnp.sum(a_sb, axis=1, keepdims=True)
    q = s_sb - p_neig
    p = jnp.where(p_neig > 0.0, p_neig, p_valid)

    loss_i = -jnp.log(p / (p + q))

    @pl.when(i == 0)
    def _init():
        loss_ref[...] = jnp.zeros_like(loss_ref)
        prec_ref[...] = jnp.zeros_like(prec_ref)

    loss_ref[...] += jnp.sum(loss_i).reshape(1, 1)
    prec_ref[...] += jnp.sum(jnp.where(loss_i < 0.6, 1.0, 0.0)).reshape(1, 1)

    @pl.when(i == G - 1)
    def _last():
        s = sim_scr[R - 1:R, :]
        pos = t_col_ref[...] == t_row[R - 1:R, :]
        lp = jnp.where(pos & (s < 1.0), 1.0, 0.0)
        ln = jnp.where(pos, 0.0, 1.0)
        mps_ref[...] = (jnp.sum(s * lp) / jnp.sum(lp)).reshape(1, 1)
        mns_ref[...] = (jnp.sum(s * ln) / jnp.sum(ln)).reshape(1, 1)
        loss_ref[...] = loss_ref[...] * (1.0 / N)
        prec_ref[...] = prec_ref[...] * (1.0 / N)


def kernel(inputs, targets):
    t32 = targets.astype(jnp.int32)
    x_rows = inputs.reshape(N, 1)
    t_rows = t32.reshape(N, 1)
    x_cols = inputs.reshape(1, N)
    t_cols = t32.reshape(1, N)

    out = pl.pallas_call(
        _nca_body,
        grid=(G,),
        in_specs=[
            pl.BlockSpec((R, 1), lambda i: (i, 0)),
            pl.BlockSpec((R, 1), lambda i: (i, 0)),
            pl.BlockSpec((1, N), lambda i: (0, 0)),
            pl.BlockSpec((1, N), lambda i: (0, 0)),
        ],
        out_specs=[
            pl.BlockSpec((1, 1), lambda i: (0, 0)),
            pl.BlockSpec((1, 1), lambda i: (0, 0)),
            pl.BlockSpec((1, 1), lambda i: (0, 0)),
            pl.BlockSpec((1, 1), lambda i: (0, 0)),
        ],
        out_shape=[jax.ShapeDtypeStruct((1, 1), jnp.float32)] * 4,
        scratch_shapes=[pltpu.VMEM((R, N), jnp.float32)],
    )(x_rows, t_rows, x_cols, t_cols)

    loss, prec, mps, mns = out
    return (loss[0, 0], prec[0, 0], mps[0, 0], mns[0, 0])


# speedup vs baseline: 1.3367x; 1.3367x over previous
import jax
import jax.numpy as jnp
from jax.experimental import pallas as pl

ALPHA = 16.0
N = 4096
R = 256
G = N // R


def _nca_body(x_row_ref, t_row_ref, x_col_ref, t_col_ref,
              loss_ref, prec_ref, mps_ref, mns_ref):
    i = pl.program_id(0)

    x_row = x_row_ref[...]
    t_row = t_row_ref[...]
    x_col = x_col_ref[...]
    t_col = t_col_ref[...]

    sim = jnp.abs(x_col - x_row)
    pos = t_col == t_row
    lt1 = sim < 1.0
    excl = pos & jnp.logical_not(lt1)
    thr = jnp.max(jnp.where(excl, -1.0, sim), axis=1, keepdims=True)

    below = sim < thr
    w = jnp.exp(-ALPHA * sim)
    pn_m = (pos & lt1) & below
    selb = below & jnp.logical_not(excl)
    p = jnp.sum(jnp.where(pn_m, w, 0.0), axis=1, keepdims=True)
    s = jnp.sum(jnp.where(selb, w, 0.0), axis=1, keepdims=True)
    q = s - p

    loss_i = jnp.where(p > 0.0, -jnp.log(p / (p + q)), 0.0)

    @pl.when(i == 0)
    def _init():
        loss_ref[...] = jnp.zeros_like(loss_ref)
        prec_ref[...] = jnp.zeros_like(prec_ref)

    loss_ref[...] += jnp.sum(loss_i).reshape(1, 1)
    prec_ref[...] += jnp.sum(jnp.where(loss_i < 0.6, 1.0, 0.0)).reshape(1, 1)

    @pl.when(i == G - 1)
    def _last():
        sl = sim[R - 1:R, :]
        lp = jnp.where(pos[R - 1:R, :] & lt1[R - 1:R, :], 1.0, 0.0)
        ln = jnp.where(pos[R - 1:R, :], 0.0, 1.0)
        mps_ref[...] = (jnp.sum(sl * lp) / jnp.sum(lp)).reshape(1, 1)
        mns_ref[...] = (jnp.sum(sl * ln) / jnp.sum(ln)).reshape(1, 1)
        loss_ref[...] = loss_ref[...] * (1.0 / N)
        prec_ref[...] = prec_ref[...] * (1.0 / N)


def kernel(inputs, targets):
    t32 = targets.astype(jnp.int32)
    x_rows = inputs.reshape(N, 1)
    t_rows = t32.reshape(N, 1)
    x_cols = inputs.reshape(1, N)
    t_cols = t32.reshape(1, N)

    out = pl.pallas_call(
        _nca_body,
        grid=(G,),
        in_specs=[
            pl.BlockSpec((R, 1), lambda i: (i, 0)),
            pl.BlockSpec((R, 1), lambda i: (i, 0)),
            pl.BlockSpec((1, N), lambda i: (0, 0)),
            pl.BlockSpec((1, N), lambda i: (0, 0)),
        ],
        out_specs=[
            pl.BlockSpec((1, 1), lambda i: (0, 0)),
            pl.BlockSpec((1, 1), lambda i: (0, 0)),
            pl.BlockSpec((1, 1), lambda i: (0, 0)),
            pl.BlockSpec((1, 1), lambda i: (0, 0)),
        ],
        out_shape=[jax.ShapeDtypeStruct((1, 1), jnp.float32)] * 4,
    )(x_rows, t_rows, x_cols, t_cols)

    loss, prec, mps, mns = out
    return (loss[0, 0], prec[0, 0], mps[0, 0], mns[0, 0])


# speedup vs baseline: 1.3938x; 1.0427x over previous
import jax
import jax.numpy as jnp
from jax.experimental import pallas as pl

ALPHA = 16.0
N = 4096
R = 512
G = N // R


def _nca_body(x_row_ref, t_row_ref, x_col_ref, t_col_ref,
              loss_ref, prec_ref, mps_ref, mns_ref):
    i = pl.program_id(0)

    x_row = x_row_ref[...]
    t_row = t_row_ref[...]
    x_col = x_col_ref[...]
    t_col = t_col_ref[...]

    sim = jnp.abs(x_col - x_row)
    pos = t_col == t_row
    lt1 = sim < 1.0
    excl = pos & jnp.logical_not(lt1)
    thr = jnp.max(jnp.where(excl, -1.0, sim), axis=1, keepdims=True)

    below = sim < thr
    w = jnp.exp(-ALPHA * sim)
    pn_m = (pos & lt1) & below
    selb = below & jnp.logical_not(excl)
    p = jnp.sum(jnp.where(pn_m, w, 0.0), axis=1, keepdims=True)
    s = jnp.sum(jnp.where(selb, w, 0.0), axis=1, keepdims=True)
    q = s - p

    loss_i = jnp.where(p > 0.0, -jnp.log(p / (p + q)), 0.0)

    @pl.when(i == 0)
    def _init():
        loss_ref[...] = jnp.zeros_like(loss_ref)
        prec_ref[...] = jnp.zeros_like(prec_ref)

    loss_ref[...] += jnp.sum(loss_i).reshape(1, 1)
    prec_ref[...] += jnp.sum(jnp.where(loss_i < 0.6, 1.0, 0.0)).reshape(1, 1)

    @pl.when(i == G - 1)
    def _last():
        sl = sim[R - 1:R, :]
        lp = jnp.where(pos[R - 1:R, :] & lt1[R - 1:R, :], 1.0, 0.0)
        ln = jnp.where(pos[R - 1:R, :], 0.0, 1.0)
        mps_ref[...] = (jnp.sum(sl * lp) / jnp.sum(lp)).reshape(1, 1)
        mns_ref[...] = (jnp.sum(sl * ln) / jnp.sum(ln)).reshape(1, 1)
        loss_ref[...] = loss_ref[...] * (1.0 / N)
        prec_ref[...] = prec_ref[...] * (1.0 / N)


def kernel(inputs, targets):
    t32 = targets.astype(jnp.int32)
    x_rows = inputs.reshape(N, 1)
    t_rows = t32.reshape(N, 1)
    x_cols = inputs.reshape(1, N)
    t_cols = t32.reshape(1, N)

    out = pl.pallas_call(
        _nca_body,
        grid=(G,),
        in_specs=[
            pl.BlockSpec((R, 1), lambda i: (i, 0)),
            pl.BlockSpec((R, 1), lambda i: (i, 0)),
            pl.BlockSpec((1, N), lambda i: (0, 0)),
            pl.BlockSpec((1, N), lambda i: (0, 0)),
        ],
        out_specs=[
            pl.BlockSpec((1, 1), lambda i: (0, 0)),
            pl.BlockSpec((1, 1), lambda i: (0, 0)),
            pl.BlockSpec((1, 1), lambda i: (0, 0)),
            pl.BlockSpec((1, 1), lambda i: (0, 0)),
        ],
        out_shape=[jax.ShapeDtypeStruct((1, 1), jnp.float32)] * 4,
    )(x_rows, t_rows, x_cols, t_cols)

    loss, prec, mps, mns = out
    return (loss[0, 0], prec[0, 0], mps[0, 0], mns[0, 0])


# speedup vs baseline: 1.4043x; 1.0076x over previous
import jax
import jax.numpy as jnp
from jax.experimental import pallas as pl

ALPHA = 16.0
N = 4096
R = 1024
G = N // R


def _nca_body(x_row_ref, t_row_ref, x_col_ref, t_col_ref,
              loss_ref, prec_ref, mps_ref, mns_ref):
    i = pl.program_id(0)

    x_row = x_row_ref[...]
    t_row = t_row_ref[...]
    x_col = x_col_ref[...]
    t_col = t_col_ref[...]

    sim = jnp.abs(x_col - x_row)
    pos = t_col == t_row
    lt1 = sim < 1.0
    excl = pos & jnp.logical_not(lt1)
    thr = jnp.max(jnp.where(excl, -1.0, sim), axis=1, keepdims=True)

    below = sim < thr
    w = jnp.exp(-ALPHA * sim)
    pn_m = (pos & lt1) & below
    selb = below & jnp.logical_not(excl)
    p = jnp.sum(jnp.where(pn_m, w, 0.0), axis=1, keepdims=True)
    s = jnp.sum(jnp.where(selb, w, 0.0), axis=1, keepdims=True)
    q = s - p

    loss_i = jnp.where(p > 0.0, -jnp.log(p / (p + q)), 0.0)

    @pl.when(i == 0)
    def _init():
        loss_ref[...] = jnp.zeros_like(loss_ref)
        prec_ref[...] = jnp.zeros_like(prec_ref)

    loss_ref[...] += jnp.sum(loss_i).reshape(1, 1)
    prec_ref[...] += jnp.sum(jnp.where(loss_i < 0.6, 1.0, 0.0)).reshape(1, 1)

    @pl.when(i == G - 1)
    def _last():
        sl = sim[R - 1:R, :]
        lp = jnp.where(pos[R - 1:R, :] & lt1[R - 1:R, :], 1.0, 0.0)
        ln = jnp.where(pos[R - 1:R, :], 0.0, 1.0)
        mps_ref[...] = (jnp.sum(sl * lp) / jnp.sum(lp)).reshape(1, 1)
        mns_ref[...] = (jnp.sum(sl * ln) / jnp.sum(ln)).reshape(1, 1)
        loss_ref[...] = loss_ref[...] * (1.0 / N)
        prec_ref[...] = prec_ref[...] * (1.0 / N)


def kernel(inputs, targets):
    t32 = targets.astype(jnp.int32)
    x_rows = inputs.reshape(N, 1)
    t_rows = t32.reshape(N, 1)
    x_cols = inputs.reshape(1, N)
    t_cols = t32.reshape(1, N)

    out = pl.pallas_call(
        _nca_body,
        grid=(G,),
        in_specs=[
            pl.BlockSpec((R, 1), lambda i: (i, 0)),
            pl.BlockSpec((R, 1), lambda i: (i, 0)),
            pl.BlockSpec((1, N), lambda i: (0, 0)),
            pl.BlockSpec((1, N), lambda i: (0, 0)),
        ],
        out_specs=[
            pl.BlockSpec((1, 1), lambda i: (0, 0)),
            pl.BlockSpec((1, 1), lambda i: (0, 0)),
            pl.BlockSpec((1, 1), lambda i: (0, 0)),
            pl.BlockSpec((1, 1), lambda i: (0, 0)),
        ],
        out_shape=[jax.ShapeDtypeStruct((1, 1), jnp.float32)] * 4,
    )(x_rows, t_rows, x_cols, t_cols)

    loss, prec, mps, mns = out
    return (loss[0, 0], prec[0, 0], mps[0, 0], mns[0, 0])
